# baseline (speedup 1.0000x reference)
import functools

import jax
import jax.numpy as jnp
from jax import lax
from jax.experimental import pallas as pl
from jax.experimental.pallas import tpu as pltpu
from jax.experimental.pallas import tpu_sc as plsc

N = 32768
NUM_SEG = 128
NUM_STATS = 6
L = 16
NC, NS = 2, 16
NW = NC * NS
CHUNK = N // NW
HIST = NUM_SEG * L
RED = NUM_STATS * NUM_SEG


def _sc_body(pred_hbm, y_hbm, dates_hbm, out_hbm,
             pred_v, y_v, dates_v, hist_v):
    wid = lax.axis_index("c") * NS + lax.axis_index("s")
    base = wid * CHUNK

    pltpu.sync_copy(pred_hbm.at[pl.ds(base, CHUNK)], pred_v)
    pltpu.sync_copy(y_hbm.at[pl.ds(base, CHUNK)], y_v)
    pltpu.sync_copy(dates_hbm.at[pl.ds(base, CHUNK)], dates_v)

    lane = lax.iota(jnp.int32, L)
    zeros = jnp.zeros((L,), jnp.float32)
    ones = jnp.ones((L,), jnp.float32)

    def zero_blk(o, _):
        for u in range(8):
            hist_v[o * 8 + u, :] = zeros
        return 0
    lax.fori_loop(0, RED // 8, zero_blk, 0)

    def accum(o, _):
        for u in range(4):
            j = o * 4 + u
            p = pred_v[pl.ds(j * L, L)]
            t = y_v[pl.ds(j * L, L)]
            d = dates_v[pl.ds(j * L, L)]
            plsc.addupdate_scatter(hist_v, [d, lane], ones)
            plsc.addupdate_scatter(hist_v, [d + NUM_SEG, lane], p)
            plsc.addupdate_scatter(hist_v, [d + 2 * NUM_SEG, lane], t)
            plsc.addupdate_scatter(hist_v, [d + 3 * NUM_SEG, lane], p * p)
            plsc.addupdate_scatter(hist_v, [d + 4 * NUM_SEG, lane], t * t)
            plsc.addupdate_scatter(hist_v, [d + 5 * NUM_SEG, lane], p * t)
        return 0
    lax.fori_loop(0, (CHUNK // L) // 4, accum, 0)

    pltpu.sync_copy(hist_v, out_hbm.at[wid])


def _sc_hist(pred, y, dates):
    mesh = plsc.VectorSubcoreMesh(core_axis_name="c", subcore_axis_name="s")
    f = pl.kernel(
        _sc_body, mesh=mesh,
        out_type=jax.ShapeDtypeStruct((NW, RED, L), jnp.float32),
        compiler_params=pltpu.CompilerParams(needs_layout_passes=False),
        scratch_types=[
            pltpu.VMEM((CHUNK,), jnp.float32),
            pltpu.VMEM((CHUNK,), jnp.float32),
            pltpu.VMEM((CHUNK,), jnp.int32),
            pltpu.VMEM((RED, L), jnp.float32),
        ],
    )
    return f(pred, y, dates)


def _tc_combine_body(part_ref, skip_ref, out_ref):
    EPS = 1e-12
    t = jnp.sum(part_ref[:, :, :], axis=(0, 2))
    n = t[0:128].reshape(1, 128)
    sp = t[128:256].reshape(1, 128)
    sy = t[256:384].reshape(1, 128)
    spp = t[384:512].reshape(1, 128)
    syy = t[512:640].reshape(1, 128)
    spy = t[640:768].reshape(1, 128)
    safe_n = jnp.maximum(n, 1.0)
    safe_nm1 = jnp.maximum(n - 1.0, 1.0)
    pm = sp / safe_n
    ym = sy / safe_n
    pvar = jnp.maximum((spp - n * pm * pm) / safe_nm1, 0.0)
    yvar = jnp.maximum((syy - n * ym * ym) / safe_nm1, 0.0)
    pstd = jnp.where(pvar > 0.0, jnp.sqrt(jnp.where(pvar > 0.0, pvar, 1.0)), 0.0)
    ystd = jnp.where(yvar > 0.0, jnp.sqrt(jnp.where(yvar > 0.0, yvar, 1.0)), 0.0)
    cross = spy - n * pm * ym
    valid = (n >= skip_ref[0, 0]) & (pstd >= EPS) & (ystd >= EPS)
    denom = jnp.where(valid, n * pstd * ystd, 1.0)
    ic = jnp.where(valid, cross / denom, 0.0)
    num_valid = jnp.sum(valid.astype(jnp.float32))
    out_ref[:, :] = (-jnp.sum(ic) / num_valid).reshape(1, 1)


def _tc_combine(partials, skip):
    return pl.pallas_call(
        _tc_combine_body,
        out_shape=jax.ShapeDtypeStruct((1, 1), jnp.float32),
    )(partials, skip)


def _sc_min_body(pred_hbm, out_hbm, hist_v):
    wid = lax.axis_index("c") * NS + lax.axis_index("s")
    pltpu.sync_copy(hist_v, out_hbm.at[wid])


def kernel(pred, y, idx, skip_size):
    mesh = plsc.VectorSubcoreMesh(core_axis_name="c", subcore_axis_name="s")
    f = pl.kernel(
        _sc_min_body, mesh=mesh,
        out_type=jax.ShapeDtypeStruct((NW, RED, L), jnp.float32),
        compiler_params=pltpu.CompilerParams(needs_layout_passes=False),
        scratch_types=[pltpu.VMEM((RED, L), jnp.float32)],
    )
    return jnp.sum(f(pred))


# speedup vs baseline: 1.2354x; 1.2354x over previous
import functools

import jax
import jax.numpy as jnp
from jax import lax
from jax.experimental import pallas as pl
from jax.experimental.pallas import tpu as pltpu
from jax.experimental.pallas import tpu_sc as plsc

N = 32768
NUM_SEG = 128
NUM_STATS = 6
L = 16
NC, NS = 2, 16
NW = NC * NS
CHUNK = N // NW
HIST = NUM_SEG * L
RED = NUM_STATS * NUM_SEG


def _sc_body(pred_hbm, y_hbm, dates_hbm, out_hbm,
             pred_v, y_v, dates_v, hist_v, red_v):
    wid = lax.axis_index("c") * NS + lax.axis_index("s")
    base = wid * CHUNK

    pltpu.sync_copy(pred_hbm.at[pl.ds(base, CHUNK)], pred_v)
    pltpu.sync_copy(y_hbm.at[pl.ds(base, CHUNK)], y_v)
    pltpu.sync_copy(dates_hbm.at[pl.ds(base, CHUNK)], dates_v)

    lane = lax.iota(jnp.int32, L)
    zeros = jnp.zeros((L,), jnp.float32)
    ones = jnp.ones((L,), jnp.float32)

    def zero_blk(o, _):
        for u in range(8):
            hist_v[o * 8 + u, :] = zeros
        return 0
    lax.fori_loop(0, RED // 8, zero_blk, 0)

    def accum(o, _):
        for u in range(4):
            j = o * 4 + u
            p = pred_v[pl.ds(j * L, L)]
            t = y_v[pl.ds(j * L, L)]
            d = dates_v[pl.ds(j * L, L)]
            plsc.addupdate_scatter(hist_v, [d, lane], ones)
            plsc.addupdate_scatter(hist_v, [d + NUM_SEG, lane], p)
            plsc.addupdate_scatter(hist_v, [d + 2 * NUM_SEG, lane], t)
            plsc.addupdate_scatter(hist_v, [d + 3 * NUM_SEG, lane], p * p)
            plsc.addupdate_scatter(hist_v, [d + 4 * NUM_SEG, lane], t * t)
            plsc.addupdate_scatter(hist_v, [d + 5 * NUM_SEG, lane], p * t)
        return 0
    lax.fori_loop(0, (CHUNK // L) // 4, accum, 0)

    last = lane == (L - 1)

    @plsc.parallel_loop(0, RED, unroll=8)
    def _reduce(g):
        s = plsc.cumsum(hist_v[g, :])
        plsc.store_scatter(red_v, [jnp.full((L,), g, jnp.int32)], s,
                           mask=last)

    pltpu.sync_copy(red_v, out_hbm.at[wid])


def _sc_hist(pred, y, dates):
    mesh = plsc.VectorSubcoreMesh(core_axis_name="c", subcore_axis_name="s")
    f = pl.kernel(
        _sc_body, mesh=mesh,
        out_type=jax.ShapeDtypeStruct((NW, RED), jnp.float32),
        compiler_params=pltpu.CompilerParams(needs_layout_passes=False),
        scratch_types=[
            pltpu.VMEM((CHUNK,), jnp.float32),
            pltpu.VMEM((CHUNK,), jnp.float32),
            pltpu.VMEM((CHUNK,), jnp.int32),
            pltpu.VMEM((RED, L), jnp.float32),
            pltpu.VMEM((RED,), jnp.float32),
        ],
    )
    return f(pred, y, dates)


def _tc_combine_body(part_ref, skip_ref, out_ref):
    EPS = 1e-12
    n = jnp.sum(part_ref[:, 0:128], axis=0, keepdims=True)
    sp = jnp.sum(part_ref[:, 128:256], axis=0, keepdims=True)
    sy = jnp.sum(part_ref[:, 256:384], axis=0, keepdims=True)
    spp = jnp.sum(part_ref[:, 384:512], axis=0, keepdims=True)
    syy = jnp.sum(part_ref[:, 512:640], axis=0, keepdims=True)
    spy = jnp.sum(part_ref[:, 640:768], axis=0, keepdims=True)
    safe_n = jnp.maximum(n, 1.0)
    safe_nm1 = jnp.maximum(n - 1.0, 1.0)
    pm = sp / safe_n
    ym = sy / safe_n
    pvar = jnp.maximum((spp - n * pm * pm) / safe_nm1, 0.0)
    yvar = jnp.maximum((syy - n * ym * ym) / safe_nm1, 0.0)
    pstd = jnp.where(pvar > 0.0, jnp.sqrt(jnp.where(pvar > 0.0, pvar, 1.0)), 0.0)
    ystd = jnp.where(yvar > 0.0, jnp.sqrt(jnp.where(yvar > 0.0, yvar, 1.0)), 0.0)
    cross = spy - n * pm * ym
    valid = (n >= skip_ref[0, 0]) & (pstd >= EPS) & (ystd >= EPS)
    denom = jnp.where(valid, n * pstd * ystd, 1.0)
    ic = jnp.where(valid, cross / denom, 0.0)
    num_valid = jnp.sum(valid.astype(jnp.float32))
    out_ref[:, :] = (-jnp.sum(ic) / num_valid).reshape(1, 1)


def _tc_combine(partials, skip):
    return pl.pallas_call(
        _tc_combine_body,
        out_shape=jax.ShapeDtypeStruct((1, 1), jnp.float32),
    )(partials, skip)


def kernel(pred, y, idx, skip_size):
    dates = idx[:, 0].astype(jnp.int32)
    partials = _sc_hist(pred, y, dates)
    skip = jnp.asarray(skip_size, jnp.float32).reshape(1, 1)
    out = _tc_combine(partials, skip)
    return out[0, 0]


# speedup vs baseline: 1.3344x; 1.0801x over previous
import functools

import jax
import jax.numpy as jnp
from jax import lax
from jax.experimental import pallas as pl
from jax.experimental.pallas import tpu as pltpu
from jax.experimental.pallas import tpu_sc as plsc

N = 32768
NUM_SEG = 128
NUM_STATS = 6
L = 16
NC, NS = 2, 16
NW = NC * NS
CHUNK = N // NW
HIST = NUM_SEG * L
RED = NUM_STATS * NUM_SEG


def _sc_body(pred_hbm, y_hbm, dates_hbm, out_hbm,
             pred_v, y_v, dates_v, hist_v, red_v, sem):
    wid = lax.axis_index("c") * NS + lax.axis_index("s")
    base = wid * CHUNK

    cp_p = pltpu.async_copy(pred_hbm.at[pl.ds(base, CHUNK)], pred_v, sem)
    cp_y = pltpu.async_copy(y_hbm.at[pl.ds(base, CHUNK)], y_v, sem)
    cp_d = pltpu.async_copy(dates_hbm.at[pl.ds(base, CHUNK)], dates_v, sem)

    lane = lax.iota(jnp.int32, L)
    zeros = jnp.zeros((L,), jnp.float32)
    ones = jnp.ones((L,), jnp.float32)

    @plsc.parallel_loop(0, RED, unroll=8)
    def _zero(g):
        hist_v[g, :] = zeros

    cp_p.wait()
    cp_y.wait()
    cp_d.wait()

    @plsc.parallel_loop(0, CHUNK // L, unroll=4)
    def _accum(j):
        p = pred_v[pl.ds(j * L, L)]
        t = y_v[pl.ds(j * L, L)]
        d = dates_v[pl.ds(j * L, L)]
        plsc.addupdate_scatter(hist_v, [d, lane], ones)
        plsc.addupdate_scatter(hist_v, [d + NUM_SEG, lane], p)
        plsc.addupdate_scatter(hist_v, [d + 2 * NUM_SEG, lane], t)
        plsc.addupdate_scatter(hist_v, [d + 3 * NUM_SEG, lane], p * p)
        plsc.addupdate_scatter(hist_v, [d + 4 * NUM_SEG, lane], t * t)
        plsc.addupdate_scatter(hist_v, [d + 5 * NUM_SEG, lane], p * t)

    last = lane == (L - 1)

    @plsc.parallel_loop(0, RED, unroll=8)
    def _reduce(g):
        s = plsc.cumsum(hist_v[g, :])
        plsc.store_scatter(red_v, [jnp.full((L,), g, jnp.int32)], s,
                           mask=last)

    pltpu.sync_copy(red_v, out_hbm.at[wid])


def _sc_hist(pred, y, dates):
    mesh = plsc.VectorSubcoreMesh(core_axis_name="c", subcore_axis_name="s")
    f = pl.kernel(
        _sc_body, mesh=mesh,
        out_type=jax.ShapeDtypeStruct((NW, RED), jnp.float32),
        compiler_params=pltpu.CompilerParams(needs_layout_passes=False),
        scratch_types=[
            pltpu.VMEM((CHUNK,), jnp.float32),
            pltpu.VMEM((CHUNK,), jnp.float32),
            pltpu.VMEM((CHUNK,), jnp.int32),
            pltpu.VMEM((RED, L), jnp.float32),
            pltpu.VMEM((RED,), jnp.float32),
            pltpu.SemaphoreType.DMA,
        ],
    )
    return f(pred, y, dates)


def _tc_combine_body(part_ref, skip_ref, out_ref):
    EPS = 1e-12
    n = jnp.sum(part_ref[:, 0:128], axis=0, keepdims=True)
    sp = jnp.sum(part_ref[:, 128:256], axis=0, keepdims=True)
    sy = jnp.sum(part_ref[:, 256:384], axis=0, keepdims=True)
    spp = jnp.sum(part_ref[:, 384:512], axis=0, keepdims=True)
    syy = jnp.sum(part_ref[:, 512:640], axis=0, keepdims=True)
    spy = jnp.sum(part_ref[:, 640:768], axis=0, keepdims=True)
    safe_n = jnp.maximum(n, 1.0)
    safe_nm1 = jnp.maximum(n - 1.0, 1.0)
    pm = sp / safe_n
    ym = sy / safe_n
    pvar = jnp.maximum((spp - n * pm * pm) / safe_nm1, 0.0)
    yvar = jnp.maximum((syy - n * ym * ym) / safe_nm1, 0.0)
    pstd = jnp.where(pvar > 0.0, jnp.sqrt(jnp.where(pvar > 0.0, pvar, 1.0)), 0.0)
    ystd = jnp.where(yvar > 0.0, jnp.sqrt(jnp.where(yvar > 0.0, yvar, 1.0)), 0.0)
    cross = spy - n * pm * ym
    valid = (n >= skip_ref[0, 0]) & (pstd >= EPS) & (ystd >= EPS)
    denom = jnp.where(valid, n * pstd * ystd, 1.0)
    ic = jnp.where(valid, cross / denom, 0.0)
    num_valid = jnp.sum(valid.astype(jnp.float32))
    out_ref[:, :] = (-jnp.sum(ic) / num_valid).reshape(1, 1)


def _tc_combine(partials, skip):
    return pl.pallas_call(
        _tc_combine_body,
        out_shape=jax.ShapeDtypeStruct((1, 1), jnp.float32),
    )(partials, skip)


def kernel(pred, y, idx, skip_size):
    dates = idx[:, 0].astype(jnp.int32)
    partials = _sc_hist(pred, y, dates)
    skip = jnp.asarray(skip_size, jnp.float32).reshape(1, 1)
    out = _tc_combine(partials, skip)
    return out[0, 0]


# speedup vs baseline: 1.3749x; 1.0304x over previous
import functools

import jax
import jax.numpy as jnp
from jax import lax
from jax.experimental import pallas as pl
from jax.experimental.pallas import tpu as pltpu
from jax.experimental.pallas import tpu_sc as plsc

N = 32768
NUM_SEG = 128
NUM_STATS = 6
L = 16
NC, NS = 2, 16
NW = NC * NS
CHUNK = N // NW
HIST = NUM_SEG * L
RED = NUM_STATS * NUM_SEG


def _sc_body(pred_hbm, y_hbm, dates_hbm, out_hbm,
             pred_v, y_v, dates_v, hist_v, red_v, sem):
    wid = lax.axis_index("c") * NS + lax.axis_index("s")
    base = wid * CHUNK

    cp_p = pltpu.async_copy(pred_hbm.at[pl.ds(base, CHUNK)], pred_v, sem)
    cp_y = pltpu.async_copy(y_hbm.at[pl.ds(base, CHUNK)], y_v, sem)
    cp_d = pltpu.async_copy(dates_hbm.at[pl.ds(base, CHUNK)], dates_v, sem)

    lane = lax.iota(jnp.int32, L)
    zeros = jnp.zeros((L,), jnp.float32)
    ones = jnp.ones((L,), jnp.float32)

    @plsc.parallel_loop(0, RED // L, unroll=8)
    def _zero_red(g):
        red_v[pl.ds(g * L, L)] = zeros

    cp_p.wait()
    cp_y.wait()
    cp_d.wait()

    d_lo = jnp.min(dates_v[pl.ds(0, L)])
    d_hi = jnp.max(dates_v[pl.ds(CHUNK - L, L)])

    def zero_blk(dr, _):
        for st in range(NUM_STATS):
            hist_v[dr + st * NUM_SEG, :] = zeros
        return 0
    lax.fori_loop(d_lo, d_hi + 1, zero_blk, 0)

    @plsc.parallel_loop(0, CHUNK // L, unroll=4)
    def _accum(j):
        p = pred_v[pl.ds(j * L, L)]
        t = y_v[pl.ds(j * L, L)]
        d = dates_v[pl.ds(j * L, L)]
        plsc.addupdate_scatter(hist_v, [d, lane], ones)
        plsc.addupdate_scatter(hist_v, [d + NUM_SEG, lane], p)
        plsc.addupdate_scatter(hist_v, [d + 2 * NUM_SEG, lane], t)
        plsc.addupdate_scatter(hist_v, [d + 3 * NUM_SEG, lane], p * p)
        plsc.addupdate_scatter(hist_v, [d + 4 * NUM_SEG, lane], t * t)
        plsc.addupdate_scatter(hist_v, [d + 5 * NUM_SEG, lane], p * t)

    last = lane == (L - 1)

    def reduce_blk(dr, _):
        for st in range(NUM_STATS):
            g = dr + st * NUM_SEG
            s = plsc.cumsum(hist_v[g, :])
            plsc.store_scatter(red_v, [jnp.full((L,), g, jnp.int32)], s,
                               mask=last)
        return 0
    lax.fori_loop(d_lo, d_hi + 1, reduce_blk, 0)

    pltpu.sync_copy(red_v, out_hbm.at[wid])


def _sc_hist(pred, y, dates):
    mesh = plsc.VectorSubcoreMesh(core_axis_name="c", subcore_axis_name="s")
    f = pl.kernel(
        _sc_body, mesh=mesh,
        out_type=jax.ShapeDtypeStruct((NW, RED), jnp.float32),
        compiler_params=pltpu.CompilerParams(needs_layout_passes=False),
        scratch_types=[
            pltpu.VMEM((CHUNK,), jnp.float32),
            pltpu.VMEM((CHUNK,), jnp.float32),
            pltpu.VMEM((CHUNK,), jnp.int32),
            pltpu.VMEM((RED, L), jnp.float32),
            pltpu.VMEM((RED,), jnp.float32),
            pltpu.SemaphoreType.DMA,
        ],
    )
    return f(pred, y, dates)


def _tc_combine_body(part_ref, skip_ref, out_ref):
    EPS = 1e-12
    n = jnp.sum(part_ref[:, 0:128], axis=0, keepdims=True)
    sp = jnp.sum(part_ref[:, 128:256], axis=0, keepdims=True)
    sy = jnp.sum(part_ref[:, 256:384], axis=0, keepdims=True)
    spp = jnp.sum(part_ref[:, 384:512], axis=0, keepdims=True)
    syy = jnp.sum(part_ref[:, 512:640], axis=0, keepdims=True)
    spy = jnp.sum(part_ref[:, 640:768], axis=0, keepdims=True)
    safe_n = jnp.maximum(n, 1.0)
    safe_nm1 = jnp.maximum(n - 1.0, 1.0)
    pm = sp / safe_n
    ym = sy / safe_n
    pvar = jnp.maximum((spp - n * pm * pm) / safe_nm1, 0.0)
    yvar = jnp.maximum((syy - n * ym * ym) / safe_nm1, 0.0)
    pstd = jnp.where(pvar > 0.0, jnp.sqrt(jnp.where(pvar > 0.0, pvar, 1.0)), 0.0)
    ystd = jnp.where(yvar > 0.0, jnp.sqrt(jnp.where(yvar > 0.0, yvar, 1.0)), 0.0)
    cross = spy - n * pm * ym
    valid = (n >= skip_ref[0, 0]) & (pstd >= EPS) & (ystd >= EPS)
    denom = jnp.where(valid, n * pstd * ystd, 1.0)
    ic = jnp.where(valid, cross / denom, 0.0)
    num_valid = jnp.sum(valid.astype(jnp.float32))
    out_ref[:, :] = (-jnp.sum(ic) / num_valid).reshape(1, 1)


def _tc_combine(partials, skip):
    return pl.pallas_call(
        _tc_combine_body,
        out_shape=jax.ShapeDtypeStruct((1, 1), jnp.float32),
    )(partials, skip)


def kernel(pred, y, idx, skip_size):
    dates = idx[:, 0].astype(jnp.int32)
    partials = _sc_hist(pred, y, dates)
    skip = jnp.asarray(skip_size, jnp.float32).reshape(1, 1)
    out = _tc_combine(partials, skip)
    return out[0, 0]


# speedup vs baseline: 1.3767x; 1.0013x over previous
import jax
import jax.numpy as jnp
from jax import lax
from jax.experimental import pallas as pl
from jax.experimental.pallas import tpu as pltpu
from jax.experimental.pallas import tpu_sc as plsc

N = 32768
NUM_SEG = 128
NUM_STATS = 6
L = 16
NC, NS = 2, 16
NW = NC * NS
CHUNK = N // NW
RED = NUM_STATS * NUM_SEG


def _sc_body(pred_hbm, y_hbm, dates_hbm, out_hbm,
             pred_v, y_v, dates_v, hist_v, red_v, sem):
    wid = lax.axis_index("c") * NS + lax.axis_index("s")
    base = wid * CHUNK

    cp_p = pltpu.async_copy(pred_hbm.at[pl.ds(base, CHUNK)], pred_v, sem)
    cp_y = pltpu.async_copy(y_hbm.at[pl.ds(base, CHUNK)], y_v, sem)
    cp_d = pltpu.async_copy(dates_hbm.at[pl.ds(base, CHUNK)], dates_v, sem)

    lane = lax.iota(jnp.int32, L)
    zeros = jnp.zeros((L,), jnp.float32)
    ones = jnp.ones((L,), jnp.float32)

    @plsc.parallel_loop(0, RED // L, unroll=8)
    def _zero_red(g):
        red_v[pl.ds(g * L, L)] = zeros

    cp_p.wait()
    cp_y.wait()
    cp_d.wait()

    d_lo = jnp.min(dates_v[pl.ds(0, L)])
    d_hi = jnp.max(dates_v[pl.ds(CHUNK - L, L)])

    def zero_blk(dr, _):
        for st in range(NUM_STATS):
            hist_v[dr + st * NUM_SEG, :] = zeros
        return 0
    lax.fori_loop(d_lo, d_hi + 1, zero_blk, 0)

    @plsc.parallel_loop(0, CHUNK // L, unroll=4)
    def _accum(j):
        p = pred_v[pl.ds(j * L, L)]
        t = y_v[pl.ds(j * L, L)]
        d = dates_v[pl.ds(j * L, L)]
        plsc.addupdate_scatter(hist_v, [d, lane], ones)
        plsc.addupdate_scatter(hist_v, [d + NUM_SEG, lane], p)
        plsc.addupdate_scatter(hist_v, [d + 2 * NUM_SEG, lane], t)
        plsc.addupdate_scatter(hist_v, [d + 3 * NUM_SEG, lane], p * p)
        plsc.addupdate_scatter(hist_v, [d + 4 * NUM_SEG, lane], t * t)
        plsc.addupdate_scatter(hist_v, [d + 5 * NUM_SEG, lane], p * t)

    last = lane == (L - 1)

    def reduce_blk(dr, _):
        for st in range(NUM_STATS):
            g = dr + st * NUM_SEG
            s = plsc.cumsum(hist_v[g, :])
            plsc.store_scatter(red_v, [jnp.full((L,), g, jnp.int32)], s,
                               mask=last)
        return 0
    lax.fori_loop(d_lo, d_hi + 1, reduce_blk, 0)

    pltpu.sync_copy(red_v, out_hbm.at[wid])


def _sc_hist(pred, y, dates):
    mesh = plsc.VectorSubcoreMesh(core_axis_name="c", subcore_axis_name="s")
    f = pl.kernel(
        _sc_body, mesh=mesh,
        out_type=jax.ShapeDtypeStruct((NW, RED), jnp.float32),
        compiler_params=pltpu.CompilerParams(needs_layout_passes=False),
        scratch_types=[
            pltpu.VMEM((CHUNK,), jnp.float32),
            pltpu.VMEM((CHUNK,), jnp.float32),
            pltpu.VMEM((CHUNK,), jnp.int32),
            pltpu.VMEM((RED, L), jnp.float32),
            pltpu.VMEM((RED,), jnp.float32),
            pltpu.SemaphoreType.DMA,
        ],
    )
    return f(pred, y, dates)


def _tc_combine_body(part_ref, skip_ref, out_ref):
    EPS = 1e-12
    n = jnp.sum(part_ref[:, 0:128], axis=0, keepdims=True)
    sp = jnp.sum(part_ref[:, 128:256], axis=0, keepdims=True)
    sy = jnp.sum(part_ref[:, 256:384], axis=0, keepdims=True)
    spp = jnp.sum(part_ref[:, 384:512], axis=0, keepdims=True)
    syy = jnp.sum(part_ref[:, 512:640], axis=0, keepdims=True)
    spy = jnp.sum(part_ref[:, 640:768], axis=0, keepdims=True)
    safe_n = jnp.maximum(n, 1.0)
    safe_nm1 = jnp.maximum(n - 1.0, 1.0)
    pm = sp / safe_n
    ym = sy / safe_n
    pvar = jnp.maximum((spp - n * pm * pm) / safe_nm1, 0.0)
    yvar = jnp.maximum((syy - n * ym * ym) / safe_nm1, 0.0)
    pstd = jnp.where(pvar > 0.0, jnp.sqrt(jnp.where(pvar > 0.0, pvar, 1.0)), 0.0)
    ystd = jnp.where(yvar > 0.0, jnp.sqrt(jnp.where(yvar > 0.0, yvar, 1.0)), 0.0)
    cross = spy - n * pm * ym
    valid = (n >= skip_ref[0, 0]) & (pstd >= EPS) & (ystd >= EPS)
    denom = jnp.where(valid, n * pstd * ystd, 1.0)
    ic = jnp.where(valid, cross / denom, 0.0)
    num_valid = jnp.sum(valid.astype(jnp.float32))
    out_ref[:, :] = (-jnp.sum(ic) / num_valid).reshape(1, 1)


def _tc_combine(partials, skip):
    return pl.pallas_call(
        _tc_combine_body,
        out_shape=jax.ShapeDtypeStruct((1, 1), jnp.float32),
    )(partials, skip)


def kernel(pred, y, idx, skip_size):
    dates = idx[:, 0].astype(jnp.int32)
    partials = _sc_hist(pred, y, dates)
    skip = jnp.asarray(skip_size, jnp.float32).reshape(1, 1)
    out = _tc_combine(partials, skip)
    return out[0, 0]
